# baseline (speedup 1.0000x reference)
import functools

import jax
import jax.numpy as jnp
from jax import lax
from jax.experimental import pallas as pl
from jax.experimental.pallas import tpu as pltpu
from jax.experimental.pallas import tpu_sc as plsc

N = 10000
E = 320000
D = 128

NW = 32
CH = 100
EPT = E // NW
NCH = EPT // CH
WCH = 25
NWIN = NCH // WCH
NBUF = 3

DEG_ROWS = 10240
DEG_SPAN = DEG_ROWS // 16
ACC_ROWS = 10112
ACC_SPAN = ACC_ROWS // 16

_mesh = plsc.VectorSubcoreMesh(core_axis_name="c", subcore_axis_name="s")

_f32 = jnp.float32


@functools.partial(
    pl.kernel,
    mesh=_mesh,
    out_type=jax.ShapeDtypeStruct((2, DEG_ROWS), _f32),
    scratch_types=[
        pltpu.VMEM((NWIN, WCH, CH), jnp.int32),
        pltpu.VMEM((DEG_SPAN,), _f32),
        pltpu.VMEM_SHARED((DEG_ROWS,), _f32),
    ],
)
def _deg_kernel(dstp_hbm, out_hbm, idx_v, ones_v, deg_sh):
    c = lax.axis_index("c")
    s = lax.axis_index("s")
    wid = s * 2 + c
    def zfill(k, carry):
        ones_v[pl.ds(k * 16, 16)] = jnp.zeros((16,), _f32)
        return carry

    lax.fori_loop(0, DEG_SPAN // 16, zfill, 0)
    pltpu.sync_copy(ones_v, deg_sh.at[pl.ds(s * DEG_SPAN, DEG_SPAN)])
    pltpu.sync_copy(dstp_hbm.at[wid], idx_v)

    def ofill(k, carry):
        ones_v[pl.ds(k * 16, 16)] = jnp.ones((16,), _f32)
        return carry

    lax.fori_loop(0, CH // 16 + 1, ofill, 0)
    plsc.subcore_barrier()

    for w in range(NWIN):
        def body(j, carry):
            pltpu.sync_copy(ones_v.at[pl.ds(0, CH)],
                            deg_sh.at[idx_v.at[w, j]], add=True)
            return carry

        lax.fori_loop(0, WCH, body, 0)
    plsc.subcore_barrier()
    pltpu.sync_copy(
        deg_sh.at[pl.ds(s * DEG_SPAN, DEG_SPAN)],
        out_hbm.at[c, pl.ds(s * DEG_SPAN, DEG_SPAN)],
    )


@functools.partial(
    pl.kernel,
    mesh=_mesh,
    out_type=jax.ShapeDtypeStruct((2, ACC_ROWS, D), _f32),
    scratch_types=[
        pltpu.VMEM((WCH, CH), jnp.int32),
        pltpu.VMEM((WCH, CH), jnp.int32),
        pltpu.VMEM((NBUF, CH, D), _f32),
        pltpu.VMEM_SHARED((ACC_ROWS, D), _f32),
        pltpu.SemaphoreType.DMA,
        pltpu.SemaphoreType.DMA,
        pltpu.SemaphoreType.DMA,
        pltpu.SemaphoreType.DMA,
        pltpu.SemaphoreType.DMA,
        pltpu.SemaphoreType.DMA,
    ],
)
def _agg_kernel(g_hbm, srcp_hbm, dstp_hbm, out_hbm,
                sidx, didx, bufs, acc_sh, g0, g1, g2, s0, s1, s2):
    c = lax.axis_index("c")
    s = lax.axis_index("s")
    wid = s * 2 + c
    gsem = (g0, g1, g2)
    ssem = (s0, s1, s2)

    def zfill(k, carry):
        r = k // 8
        bufs[0, r, pl.ds((k % 8) * 16, 16)] = jnp.zeros((16,), _f32)
        return carry

    lax.fori_loop(0, CH * 8, zfill, 0)
    base = s * ACC_SPAN
    for t in range(6):
        pltpu.sync_copy(bufs.at[0, pl.ds(0, 96)],
                        acc_sh.at[pl.ds(base + t * 96, 96)])
    pltpu.sync_copy(bufs.at[0, pl.ds(0, 56)],
                    acc_sh.at[pl.ds(base + 576, 56)])
    plsc.subcore_barrier()

    pltpu.sync_copy(srcp_hbm.at[wid, 0], sidx)
    pltpu.sync_copy(dstp_hbm.at[wid, 0], didx)

    gh = {0: pltpu.async_copy(g_hbm.at[sidx.at[0]], bufs.at[0], gsem[0])}
    sh = {}
    drained = -1
    for j in range(NCH):
        b = j % NBUF
        gh.pop(j).wait()
        sh[j] = pltpu.async_copy(bufs.at[b], acc_sh.at[didx.at[j % WCH]],
                                 ssem[b], add=True)
        nj = j + 1
        if nj == NCH:
            break
        if nj % WCH == 0:
            while drained < j:
                drained += 1
                sh.pop(drained).wait()
            w = nj // WCH
            pltpu.sync_copy(srcp_hbm.at[wid, w], sidx)
            pltpu.sync_copy(dstp_hbm.at[wid, w], didx)
        nb = nj % NBUF
        if nj - NBUF > drained:
            drained = nj - NBUF
            sh.pop(drained).wait()
        gh[nj] = pltpu.async_copy(g_hbm.at[sidx.at[nj % WCH]], bufs.at[nb],
                                  gsem[nb])
    while drained < NCH - 1:
        drained += 1
        sh.pop(drained).wait()

    plsc.subcore_barrier()
    pltpu.sync_copy(
        acc_sh.at[pl.ds(s * ACC_SPAN, ACC_SPAN)],
        out_hbm.at[c, pl.ds(s * ACC_SPAN, ACC_SPAN), :],
    )


_BR = 1000


def _g_body(x_ref, w_ref, degt_ref, g_ref, dis_ref):
    deg = jnp.sum(degt_ref[...], axis=1, keepdims=True) + 1.0
    dis = lax.rsqrt(deg)
    h = jnp.dot(x_ref[...], w_ref[...], preferred_element_type=_f32)
    g_ref[...] = h * dis
    dis_ref[...] = dis


_g_call = pl.pallas_call(
    _g_body,
    grid=(N // _BR,),
    in_specs=[
        pl.BlockSpec((_BR, D), lambda i: (i, 0)),
        pl.BlockSpec((D, D), lambda i: (0, 0)),
        pl.BlockSpec((_BR, 2), lambda i: (i, 0)),
    ],
    out_specs=[
        pl.BlockSpec((_BR, D), lambda i: (i, 0)),
        pl.BlockSpec((_BR, 1), lambda i: (i, 0)),
    ],
    out_shape=[
        jax.ShapeDtypeStruct((N, D), _f32),
        jax.ShapeDtypeStruct((N, 1), _f32),
    ],
)


def _fin_body(acc_ref, g_ref, dis_ref, b_ref, o_ref):
    t = acc_ref[0] + acc_ref[1] + g_ref[...]
    o_ref[...] = jnp.maximum(t * dis_ref[...] + b_ref[...], 0.0)


_fin_call = pl.pallas_call(
    _fin_body,
    grid=(N // _BR,),
    in_specs=[
        pl.BlockSpec((2, _BR, D), lambda i: (0, i, 0)),
        pl.BlockSpec((_BR, D), lambda i: (i, 0)),
        pl.BlockSpec((_BR, 1), lambda i: (i, 0)),
        pl.BlockSpec((1, D), lambda i: (0, 0)),
    ],
    out_specs=pl.BlockSpec((_BR, D), lambda i: (i, 0)),
    out_shape=jax.ShapeDtypeStruct((N, D), _f32),
)


def kernel(x, edge_index, W, b):
    srcp = edge_index[0].reshape(NW, NWIN, WCH, CH)
    dstp = edge_index[1].reshape(NW, NWIN, WCH, CH)

    deg_parts = _deg_kernel(dstp)
    degt = deg_parts[:, :N].T

    g, dis = _g_call(x, W, degt)

    acc_parts = _agg_kernel(g, srcp, dstp)

    return _fin_call(acc_parts, g, dis, b.reshape(1, D))


# speedup vs baseline: 1.1764x; 1.1764x over previous
import functools

import jax
import jax.numpy as jnp
from jax import lax
from jax.experimental import pallas as pl
from jax.experimental.pallas import tpu as pltpu
from jax.experimental.pallas import tpu_sc as plsc

N = 10000
E = 320000
D = 128

NW = 32
CH = 125
EPT = E // NW
NCH = EPT // CH
WCH = 16
NWIN = NCH // WCH

DEG_ROWS = 10240
DEG_SPAN = DEG_ROWS // 16
ACC_ROWS = 10112
ACC_SPAN = ACC_ROWS // 16

_mesh = plsc.VectorSubcoreMesh(core_axis_name="c", subcore_axis_name="s")

_f32 = jnp.float32


@functools.partial(
    pl.kernel,
    mesh=_mesh,
    out_type=jax.ShapeDtypeStruct((2, DEG_ROWS), _f32),
    scratch_types=[
        pltpu.VMEM((NWIN, WCH, CH), jnp.int32),
        pltpu.VMEM((DEG_SPAN,), _f32),
        pltpu.VMEM_SHARED((DEG_ROWS,), _f32),
    ],
)
def _deg_kernel(dstp_hbm, out_hbm, idx_v, ones_v, deg_sh):
    c = lax.axis_index("c")
    s = lax.axis_index("s")
    wid = s * 2 + c
    def zfill(k, carry):
        ones_v[pl.ds(k * 16, 16)] = jnp.zeros((16,), _f32)
        return carry

    lax.fori_loop(0, DEG_SPAN // 16, zfill, 0)
    pltpu.sync_copy(ones_v, deg_sh.at[pl.ds(s * DEG_SPAN, DEG_SPAN)])
    pltpu.sync_copy(dstp_hbm.at[wid], idx_v)

    def ofill(k, carry):
        ones_v[pl.ds(k * 16, 16)] = jnp.ones((16,), _f32)
        return carry

    lax.fori_loop(0, CH // 16 + 1, ofill, 0)
    plsc.subcore_barrier()

    for w in range(NWIN):
        def body(j, carry):
            pltpu.sync_copy(ones_v.at[pl.ds(0, CH)],
                            deg_sh.at[idx_v.at[w, j]], add=True)
            return carry

        lax.fori_loop(0, WCH, body, 0)
    plsc.subcore_barrier()
    pltpu.sync_copy(
        deg_sh.at[pl.ds(s * DEG_SPAN, DEG_SPAN)],
        out_hbm.at[c, pl.ds(s * DEG_SPAN, DEG_SPAN)],
    )


@functools.partial(
    pl.kernel,
    mesh=_mesh,
    out_type=jax.ShapeDtypeStruct((2, ACC_ROWS, D), _f32),
    scratch_types=[
        pltpu.VMEM((2, WCH, CH), jnp.int32),
        pltpu.VMEM((2, WCH, CH), jnp.int32),
        pltpu.VMEM((CH, D), _f32),
        pltpu.VMEM((CH, D), _f32),
        pltpu.VMEM_SHARED((ACC_ROWS, D), _f32),
        pltpu.SemaphoreType.DMA,
        pltpu.SemaphoreType.DMA,
        pltpu.SemaphoreType.DMA,
    ],
)
def _agg_kernel(g_hbm, srcp_hbm, dstp_hbm, out_hbm,
                sidx, didx, buf0, buf1, acc_sh, sem0, sem1, wsem):
    c = lax.axis_index("c")
    s = lax.axis_index("s")
    wid = s * 2 + c

    def zfill(k, carry):
        r = k // 8
        buf0[r, pl.ds((k % 8) * 16, 16)] = jnp.zeros((16,), _f32)
        return carry

    lax.fori_loop(0, CH * 8, zfill, 0)
    base = s * ACC_SPAN
    for t in range(5):
        pltpu.sync_copy(buf0.at[pl.ds(0, 120)],
                        acc_sh.at[pl.ds(base + t * 120, 120)])
    pltpu.sync_copy(buf0.at[pl.ds(0, 32)],
                    acc_sh.at[pl.ds(base + 600, 32)])
    plsc.subcore_barrier()

    pltpu.sync_copy(srcp_hbm.at[wid, 0], sidx.at[0])
    pltpu.sync_copy(dstp_hbm.at[wid, 0], didx.at[0])
    wh = [pltpu.async_copy(srcp_hbm.at[wid, 1], sidx.at[1], wsem),
          pltpu.async_copy(dstp_hbm.at[wid, 1], didx.at[1], wsem)]
    for w in range(NWIN):
        sl = w % 2
        if w > 0:
            for h in wh:
                h.wait()
            wh = []
        if w + 1 < NWIN:
            wh = [pltpu.async_copy(srcp_hbm.at[wid, w + 1],
                                   sidx.at[(w + 1) % 2], wsem),
                  pltpu.async_copy(dstp_hbm.at[wid, w + 1],
                                   didx.at[(w + 1) % 2], wsem)]

        pltpu.async_copy(g_hbm.at[sidx.at[sl, 0]], buf0, sem0)

        def body(i, carry):
            j0 = 2 * i
            pltpu.async_copy(g_hbm.at[sidx.at[sl, j0 + 1]], buf1, sem1)
            pltpu.make_async_copy(g_hbm.at[sidx.at[sl, j0]], buf0,
                                  sem0).wait()
            pltpu.sync_copy(buf0, acc_sh.at[didx.at[sl, j0]], add=True)

            @pl.when(i < WCH // 2 - 1)
            def _():
                pltpu.async_copy(g_hbm.at[sidx.at[sl, j0 + 2]], buf0, sem0)

            pltpu.make_async_copy(g_hbm.at[sidx.at[sl, j0]], buf1,
                                  sem1).wait()
            pltpu.sync_copy(buf1, acc_sh.at[didx.at[sl, j0 + 1]], add=True)
            return carry

        lax.fori_loop(0, WCH // 2, body, 0)

    plsc.subcore_barrier()
    pltpu.sync_copy(
        acc_sh.at[pl.ds(s * ACC_SPAN, ACC_SPAN)],
        out_hbm.at[c, pl.ds(s * ACC_SPAN, ACC_SPAN), :],
    )


_BR = 1000


def _g_body(x_ref, w_ref, degt_ref, g_ref, dis_ref):
    deg = jnp.sum(degt_ref[...], axis=1, keepdims=True) + 1.0
    dis = lax.rsqrt(deg)
    h = jnp.dot(x_ref[...], w_ref[...], preferred_element_type=_f32)
    g_ref[...] = h * dis
    dis_ref[...] = dis


_g_call = pl.pallas_call(
    _g_body,
    grid=(N // _BR,),
    in_specs=[
        pl.BlockSpec((_BR, D), lambda i: (i, 0)),
        pl.BlockSpec((D, D), lambda i: (0, 0)),
        pl.BlockSpec((_BR, 2), lambda i: (i, 0)),
    ],
    out_specs=[
        pl.BlockSpec((_BR, D), lambda i: (i, 0)),
        pl.BlockSpec((_BR, 1), lambda i: (i, 0)),
    ],
    out_shape=[
        jax.ShapeDtypeStruct((N, D), _f32),
        jax.ShapeDtypeStruct((N, 1), _f32),
    ],
)


def _fin_body(acc_ref, g_ref, dis_ref, b_ref, o_ref):
    t = acc_ref[0] + acc_ref[1] + g_ref[...]
    o_ref[...] = jnp.maximum(t * dis_ref[...] + b_ref[...], 0.0)


_fin_call = pl.pallas_call(
    _fin_body,
    grid=(N // _BR,),
    in_specs=[
        pl.BlockSpec((2, _BR, D), lambda i: (0, i, 0)),
        pl.BlockSpec((_BR, D), lambda i: (i, 0)),
        pl.BlockSpec((_BR, 1), lambda i: (i, 0)),
        pl.BlockSpec((1, D), lambda i: (0, 0)),
    ],
    out_specs=pl.BlockSpec((_BR, D), lambda i: (i, 0)),
    out_shape=jax.ShapeDtypeStruct((N, D), _f32),
)


def kernel(x, edge_index, W, b):
    srcp = edge_index[0].reshape(NW, NWIN, WCH, CH)
    dstp = edge_index[1].reshape(NW, NWIN, WCH, CH)

    deg_parts = _deg_kernel(dstp)
    degt = deg_parts[:, :N].T

    g, dis = _g_call(x, W, degt)

    acc_parts = _agg_kernel(g, srcp, dstp)

    return _fin_call(acc_parts, g, dis, b.reshape(1, D))
